# baseline (speedup 1.0000x reference)
import functools

import jax
import jax.numpy as jnp
import numpy as np
from jax.experimental import pallas as pl
from jax.experimental.pallas import tpu as pltpu

DIM = 2048
HEADS = 16
DH = 128
INNER = HEADS * DH
EPS = 1e-05
B = 2
S = 2048
M = B * S
MT = 512
BQ = 512
KC = 128
SCALE = 1.0 / np.sqrt(DH)
C1 = SCALE * np.log2(np.e)

_TRANS_B = (((1,), (1,)), ((), ()))


def _qkv_kernel(x_ref, wq_ref, wk_ref, wv_ref, bq_ref, bk_ref,
                gq_ref, gk_ref, o_ref):
    x = x_ref[...].astype(jnp.bfloat16)
    q = jax.lax.dot_general(x, wq_ref[...], _TRANS_B,
                            preferred_element_type=jnp.float32) + bq_ref[...]
    k = jax.lax.dot_general(x, wk_ref[...], _TRANS_B,
                            preferred_element_type=jnp.float32) + bk_ref[...]
    v = jax.lax.dot_general(x, wv_ref[...], _TRANS_B,
                            preferred_element_type=jnp.float32)
    qv = jnp.mean(q * q, axis=-1, keepdims=True)
    qn = (q * (jax.lax.rsqrt(qv + EPS) * C1) * gq_ref[...]).astype(jnp.bfloat16)
    kv = jnp.mean(k * k, axis=-1, keepdims=True)
    kn = (k * jax.lax.rsqrt(kv + EPS) * gk_ref[...]).astype(jnp.bfloat16)
    o_ref[...] = jnp.concatenate([qn, kn, v.astype(jnp.bfloat16)], axis=-1)


def _attn_kernel(q_ref, k_ref, v_ref, wo_ref, bo_ref, o_ref):
    outs = []
    for h in range(HEADS):
        sl = slice(h * DH, (h + 1) * DH)
        q = q_ref[0][:, sl]
        vh = v_ref[0][:, sl]
        vx = jnp.concatenate([vh, jnp.ones_like(vh)], axis=-1)
        m = jnp.full((BQ, 1), -1e30, jnp.float32)
        acc = jnp.zeros((BQ, 2 * DH), jnp.float32)
        for j in range(S // KC):
            k = k_ref[0][j * KC:(j + 1) * KC, sl]
            s = jax.lax.dot_general(q, k, _TRANS_B,
                                    preferred_element_type=jnp.float32)
            mn = jnp.maximum(m, jnp.max(s, axis=-1, keepdims=True))
            p = jnp.exp2(s - mn)
            alpha = jnp.exp2(m - mn)
            pv = jax.lax.dot_general(p.astype(jnp.bfloat16),
                                     vx[j * KC:(j + 1) * KC, :],
                                     (((1,), (0,)), ((), ())),
                                     preferred_element_type=jnp.float32)
            acc = acc * alpha + pv
            m = mn
        outs.append((acc[:, :DH] / acc[:, DH:]).astype(jnp.bfloat16))
    o = jnp.concatenate(outs, axis=-1)
    proj = jax.lax.dot_general(o, wo_ref[...], _TRANS_B,
                               preferred_element_type=jnp.float32)
    o_ref[0] = proj + bo_ref[...]


def kernel(x, Wq, bq, Wk, bk, Wv, bv, gq, gk, Wo, bo):
    x2 = x.reshape(M, DIM)
    wqb = Wq.astype(jnp.bfloat16)
    wkb = Wk.astype(jnp.bfloat16)
    wvb = Wv.astype(jnp.bfloat16)
    wob = Wo.astype(jnp.bfloat16)
    bq2 = bq.reshape(1, INNER)
    bk2 = bk.reshape(1, INNER)
    gq2 = gq.reshape(1, INNER)
    gk2 = gk.reshape(1, INNER)
    bo2 = (bo + bv @ Wo.T).reshape(1, DIM)

    full = pl.BlockSpec((INNER, DIM), lambda *a: (0, 0))
    row = pl.BlockSpec((1, INNER), lambda *a: (0, 0))
    tok = pl.BlockSpec((MT, DIM), lambda i: (i, 0))

    qkv = pl.pallas_call(
        _qkv_kernel,
        grid=(M // MT,),
        in_specs=[tok, full, full, full, row, row, row, row],
        out_specs=pl.BlockSpec((MT, 3 * INNER), lambda i: (i, 0)),
        out_shape=jax.ShapeDtypeStruct((M, 3 * INNER), jnp.bfloat16),
    )(x2, wqb, wkb, wvb, bq2, bk2, gq2, gk2)

    qkv = qkv.reshape(B, S, 3 * INNER)

    qspec = pl.BlockSpec((1, BQ, INNER), lambda b, qi: (b, qi, 0))
    kspec = pl.BlockSpec((1, S, INNER), lambda b, qi: (b, 0, 1))
    vspec = pl.BlockSpec((1, S, INNER), lambda b, qi: (b, 0, 2))
    out = pl.pallas_call(
        _attn_kernel,
        grid=(B, S // BQ),
        in_specs=[qspec, kspec, vspec,
                  pl.BlockSpec((DIM, INNER), lambda *a: (0, 0)),
                  pl.BlockSpec((1, DIM), lambda *a: (0, 0))],
        out_specs=pl.BlockSpec((1, BQ, DIM), lambda b, qi: (b, qi, 0)),
        out_shape=jax.ShapeDtypeStruct((B, S, DIM), jnp.float32),
    )(qkv, qkv, qkv, wob, bo2)

    return out


# speedup vs baseline: 1.9262x; 1.9262x over previous
import jax
import jax.numpy as jnp
import numpy as np
from jax.experimental import pallas as pl

DIM = 2048
HEADS = 16
DH = 128
INNER = HEADS * DH
EPS = 1e-05
B = 2
S = 2048
M = B * S
MT = 512
BQ = 512
KC = 256
SCALE = 1.0 / np.sqrt(DH)
C1 = SCALE * np.log2(np.e)

_TRANS_B = (((1,), (1,)), ((), ()))


def _qkv_kernel(x_ref, wq_ref, wk_ref, wv_ref, bq_ref, bk_ref,
                gq_ref, gk_ref, o_ref):
    x = x_ref[...].astype(jnp.bfloat16)
    q = jax.lax.dot_general(x, wq_ref[...], _TRANS_B,
                            preferred_element_type=jnp.float32) + bq_ref[...]
    k = jax.lax.dot_general(x, wk_ref[...], _TRANS_B,
                            preferred_element_type=jnp.float32) + bk_ref[...]
    v = jax.lax.dot_general(x, wv_ref[...], _TRANS_B,
                            preferred_element_type=jnp.float32)
    qv = jnp.mean(q * q, axis=-1, keepdims=True)
    qn = (q * (jax.lax.rsqrt(qv + EPS) * C1) * gq_ref[...]).astype(jnp.bfloat16)
    kv = jnp.mean(k * k, axis=-1, keepdims=True)
    kn = (k * jax.lax.rsqrt(kv + EPS) * gk_ref[...]).astype(jnp.bfloat16)
    o_ref[...] = jnp.concatenate([qn, kn, v.astype(jnp.bfloat16)], axis=-1)


def _attn_kernel(q_ref, k_ref, v_ref, wo_ref, bo_ref, o_ref):
    outs = []
    for h in range(HEADS):
        sl = slice(h * DH, (h + 1) * DH)
        q = q_ref[0][:, sl]
        vh = v_ref[0][:, sl]
        vx = jnp.concatenate([vh, jnp.ones_like(vh)], axis=-1)
        m = jnp.full((BQ, 1), -1e30, jnp.float32)
        acc = jnp.zeros((BQ, 2 * DH), jnp.float32)
        for j in range(S // KC):
            k = k_ref[0][j * KC:(j + 1) * KC, sl]
            s = jax.lax.dot_general(q, k, _TRANS_B,
                                    preferred_element_type=jnp.float32)
            mn = jnp.maximum(m, jnp.max(s, axis=-1, keepdims=True))
            p = jnp.exp2(s - mn)
            alpha = jnp.exp2(m - mn)
            pv = jax.lax.dot_general(p.astype(jnp.bfloat16),
                                     vx[j * KC:(j + 1) * KC, :],
                                     (((1,), (0,)), ((), ())),
                                     preferred_element_type=jnp.float32)
            acc = acc * alpha + pv
            m = mn
        outs.append((acc[:, :DH] / acc[:, DH:]).astype(jnp.bfloat16))
    o = jnp.concatenate(outs, axis=-1)
    proj = jax.lax.dot_general(o, wo_ref[...], _TRANS_B,
                               preferred_element_type=jnp.float32)
    o_ref[0] = proj + bo_ref[...]


def kernel(x, Wq, bq, Wk, bk, Wv, bv, gq, gk, Wo, bo):
    x2 = x.reshape(M, DIM)
    wqb = Wq.astype(jnp.bfloat16)
    wkb = Wk.astype(jnp.bfloat16)
    wvb = Wv.astype(jnp.bfloat16)
    wob = Wo.astype(jnp.bfloat16)
    bq2 = bq.reshape(1, INNER)
    bk2 = bk.reshape(1, INNER)
    gq2 = gq.reshape(1, INNER)
    gk2 = gk.reshape(1, INNER)
    bo2 = (bo + bv @ Wo.T).reshape(1, DIM)

    full = pl.BlockSpec((INNER, DIM), lambda *a: (0, 0))
    row = pl.BlockSpec((1, INNER), lambda *a: (0, 0))
    tok = pl.BlockSpec((MT, DIM), lambda i: (i, 0))

    qkv = pl.pallas_call(
        _qkv_kernel,
        grid=(M // MT,),
        in_specs=[tok, full, full, full, row, row, row, row],
        out_specs=pl.BlockSpec((MT, 3 * INNER), lambda i: (i, 0)),
        out_shape=jax.ShapeDtypeStruct((M, 3 * INNER), jnp.bfloat16),
    )(x2, wqb, wkb, wvb, bq2, bk2, gq2, gk2)

    qkv = qkv.reshape(B, S, 3 * INNER)

    qspec = pl.BlockSpec((1, BQ, INNER), lambda b, qi: (b, qi, 0))
    kspec = pl.BlockSpec((1, S, INNER), lambda b, qi: (b, 0, 1))
    vspec = pl.BlockSpec((1, S, INNER), lambda b, qi: (b, 0, 2))
    out = pl.pallas_call(
        _attn_kernel,
        grid=(B, S // BQ),
        in_specs=[qspec, kspec, vspec,
                  pl.BlockSpec((DIM, INNER), lambda *a: (0, 0)),
                  pl.BlockSpec((1, DIM), lambda *a: (0, 0))],
        out_specs=pl.BlockSpec((1, BQ, DIM), lambda b, qi: (b, qi, 0)),
        out_shape=jax.ShapeDtypeStruct((B, S, DIM), jnp.float32),
    )(qkv, qkv, qkv, wob, bo2)

    return out


# speedup vs baseline: 1.9903x; 1.0333x over previous
import jax
import jax.numpy as jnp
import numpy as np
from jax.experimental import pallas as pl

DIM = 2048
HEADS = 16
DH = 128
INNER = HEADS * DH
EPS = 1e-05
B = 2
S = 2048
M = B * S
MT = 256
BQ = 512
KC = 256
SCALE = 1.0 / np.sqrt(DH)
C1 = SCALE * np.log2(np.e)

_TRANS_B = (((1,), (1,)), ((), ()))


def _qkv_kernel(x_ref, wq_ref, wk_ref, wv_ref, bq_ref, bk_ref,
                gq_ref, gk_ref, o_ref):
    x = x_ref[...].astype(jnp.bfloat16)
    q = jax.lax.dot_general(x, wq_ref[...], _TRANS_B,
                            preferred_element_type=jnp.float32) + bq_ref[...]
    k = jax.lax.dot_general(x, wk_ref[...], _TRANS_B,
                            preferred_element_type=jnp.float32) + bk_ref[...]
    v = jax.lax.dot_general(x, wv_ref[...], _TRANS_B,
                            preferred_element_type=jnp.float32)
    qv = jnp.mean(q * q, axis=-1, keepdims=True)
    qn = (q * (jax.lax.rsqrt(qv + EPS) * C1) * gq_ref[...]).astype(jnp.bfloat16)
    kv = jnp.mean(k * k, axis=-1, keepdims=True)
    kn = (k * jax.lax.rsqrt(kv + EPS) * gk_ref[...]).astype(jnp.bfloat16)
    o_ref[...] = jnp.concatenate([qn, kn, v.astype(jnp.bfloat16)], axis=-1)


def _attn_kernel(q_ref, k_ref, v_ref, wo_ref, bo_ref, o_ref):
    outs = []
    for h in range(HEADS):
        sl = slice(h * DH, (h + 1) * DH)
        q = q_ref[0][:, sl]
        vh = v_ref[0][:, sl]
        vx = jnp.concatenate([vh, jnp.ones_like(vh)], axis=-1)
        m = jnp.full((BQ, 1), -1e30, jnp.float32)
        acc = jnp.zeros((BQ, 2 * DH), jnp.float32)
        for j in range(S // KC):
            k = k_ref[0][j * KC:(j + 1) * KC, sl]
            s = jax.lax.dot_general(q, k, _TRANS_B,
                                    preferred_element_type=jnp.float32)
            mn = jnp.maximum(m, jnp.max(s, axis=-1, keepdims=True))
            p = jnp.exp2(s - mn)
            alpha = jnp.exp2(m - mn)
            pv = jax.lax.dot_general(p.astype(jnp.bfloat16),
                                     vx[j * KC:(j + 1) * KC, :],
                                     (((1,), (0,)), ((), ())),
                                     preferred_element_type=jnp.float32)
            acc = acc * alpha + pv
            m = mn
        outs.append((acc[:, :DH] / acc[:, DH:]).astype(jnp.bfloat16))
    o = jnp.concatenate(outs, axis=-1)
    proj = jax.lax.dot_general(o, wo_ref[...], _TRANS_B,
                               preferred_element_type=jnp.float32)
    o_ref[0] = proj + bo_ref[...]


def kernel(x, Wq, bq, Wk, bk, Wv, bv, gq, gk, Wo, bo):
    x2 = x.reshape(M, DIM)
    wqb = Wq.astype(jnp.bfloat16)
    wkb = Wk.astype(jnp.bfloat16)
    wvb = Wv.astype(jnp.bfloat16)
    wob = Wo.astype(jnp.bfloat16)
    bq2 = bq.reshape(1, INNER)
    bk2 = bk.reshape(1, INNER)
    gq2 = gq.reshape(1, INNER)
    gk2 = gk.reshape(1, INNER)
    bo2 = (bo + bv @ Wo.T).reshape(1, DIM)

    full = pl.BlockSpec((INNER, DIM), lambda *a: (0, 0))
    row = pl.BlockSpec((1, INNER), lambda *a: (0, 0))
    tok = pl.BlockSpec((MT, DIM), lambda i: (i, 0))

    qkv = pl.pallas_call(
        _qkv_kernel,
        grid=(M // MT,),
        in_specs=[tok, full, full, full, row, row, row, row],
        out_specs=pl.BlockSpec((MT, 3 * INNER), lambda i: (i, 0)),
        out_shape=jax.ShapeDtypeStruct((M, 3 * INNER), jnp.bfloat16),
    )(x2, wqb, wkb, wvb, bq2, bk2, gq2, gk2)

    qkv = qkv.reshape(B, S, 3 * INNER)

    qspec = pl.BlockSpec((1, BQ, INNER), lambda b, qi: (b, qi, 0))
    kspec = pl.BlockSpec((1, S, INNER), lambda b, qi: (b, 0, 1))
    vspec = pl.BlockSpec((1, S, INNER), lambda b, qi: (b, 0, 2))
    out = pl.pallas_call(
        _attn_kernel,
        grid=(B, S // BQ),
        in_specs=[qspec, kspec, vspec,
                  pl.BlockSpec((DIM, INNER), lambda *a: (0, 0)),
                  pl.BlockSpec((1, DIM), lambda *a: (0, 0))],
        out_specs=pl.BlockSpec((1, BQ, DIM), lambda b, qi: (b, qi, 0)),
        out_shape=jax.ShapeDtypeStruct((B, S, DIM), jnp.float32),
    )(qkv, qkv, qkv, wob, bo2)

    return out
